# baseline (speedup 1.0000x reference)
import jax
import jax.numpy as jnp
from jax import lax
from jax.experimental import pallas as pl
from jax.experimental.pallas import tpu as pltpu
from jax.experimental.pallas import tpu_sc as plsc

B = 16384
D = 64
ROWS_PER_TILE = 8
NUM_TAGS = 64
ITEMS_PER_TAG = 4
NC = 2
NS = 16
NW = NC * NS
B_PER_W = B // NW
C = NUM_TAGS * ITEMS_PER_TAG
C_PER_W = C // NW
K = 32
NCH = B_PER_W // K


def _sc_gather_body(user_w, item_w, user_idx, pos_idx, neg_idx,
                    cidx, user_out, pos_out, neg_out, crows_out,
                    iu_v, ip_v, in_v, ic_v, obuf0, obuf1, cobuf,
                    rsem0, rsem1, osem0, osem1, csem):
    wid = lax.axis_index("s") * NC + lax.axis_index("c")
    base = wid * B_PER_W
    cbase = wid * C_PER_W
    obufs = (obuf0, obuf1)
    rsems = (rsem0, rsem1)
    osems = (osem0, osem1)

    pltpu.sync_copy(user_idx.at[pl.ds(base, B_PER_W)], iu_v)
    pltpu.sync_copy(pos_idx.at[pl.ds(base, B_PER_W)], ip_v)
    pltpu.sync_copy(neg_idx.at[pl.ds(base, B_PER_W)], in_v)
    pltpu.sync_copy(cidx.at[pl.ds(cbase, C_PER_W)], ic_v.at[pl.ds(0, C_PER_W)])

    civ = ic_v[...]
    for jj in range(C_PER_W):
        pltpu.async_copy(item_w.at[civ[jj]], cobuf.at[jj], csem)

    def fire(table, idx_v, c, par):
        for g in range(K // 16):
            iv = idx_v[pl.ds(c * K + g * 16, 16)]
            for jj in range(16):
                pltpu.async_copy(table.at[iv[jj]],
                                 obufs[par].at[g * 16 + jj], rsems[par])

    def drain_rows(par):
        for g in range(K // ROWS_PER_TILE):
            pltpu.make_async_copy(
                user_w.at[pl.ds(0, ROWS_PER_TILE)],
                obufs[par].at[pl.ds(g * ROWS_PER_TILE, ROWS_PER_TILE)],
                rsems[par]).wait()

    def out_copy(out, c, par):
        return pltpu.async_copy(
            obufs[par], out.at[pl.ds(base + c * K, K)], osems[par])

    def out_drain(out, c, par):
        pltpu.make_async_copy(
            obufs[par], out.at[pl.ds(base + c * K, K)], osems[par]).wait()

    for table, idx_v, out in (
        (user_w, iu_v, user_out),
        (item_w, ip_v, pos_out),
        (item_w, in_v, neg_out),
    ):
        fire(table, idx_v, 0, 0)

        def body(c2, _, table=table, idx_v=idx_v, out=out):
            for par in range(2):
                c = c2 * 2 + par

                @pl.when(c + 1 < NCH)
                def _():
                    @pl.when(c >= 1)
                    def _():
                        out_drain(out, c - 1, 1 - par)

                    fire(table, idx_v, c + 1, 1 - par)

                drain_rows(par)
                out_copy(out, c, par)
            return 0

        lax.fori_loop(0, NCH // 2, body, 0)
        out_drain(out, NCH - 2, 0)
        out_drain(out, NCH - 1, 1)

    pltpu.make_async_copy(
        user_w.at[pl.ds(0, C_PER_W)], cobuf, csem).wait()
    pltpu.sync_copy(cobuf, crows_out.at[pl.ds(cbase, C_PER_W)])


_sc_gather = pl.kernel(
    _sc_gather_body,
    out_type=(
        jax.ShapeDtypeStruct((B, D), jnp.float32),
        jax.ShapeDtypeStruct((B, D), jnp.float32),
        jax.ShapeDtypeStruct((B, D), jnp.float32),
        jax.ShapeDtypeStruct((C, D), jnp.float32),
    ),
    mesh=plsc.VectorSubcoreMesh(core_axis_name="c", subcore_axis_name="s"),
    scratch_types=[
        pltpu.VMEM((B_PER_W,), jnp.int32),
        pltpu.VMEM((B_PER_W,), jnp.int32),
        pltpu.VMEM((B_PER_W,), jnp.int32),
        pltpu.VMEM((16,), jnp.int32),
        pltpu.VMEM((K, D), jnp.float32),
        pltpu.VMEM((K, D), jnp.float32),
        pltpu.VMEM((C_PER_W, D), jnp.float32),
        pltpu.SemaphoreType.DMA,
        pltpu.SemaphoreType.DMA,
        pltpu.SemaphoreType.DMA,
        pltpu.SemaphoreType.DMA,
        pltpu.SemaphoreType.DMA,
    ],
)


BLK = 2048


def _tc_sim_body(pos_ref, neg_ref, cw_ref, cv_ref, pos_out, neg_out):
    cv = jnp.sum(cw_ref[...] * cv_ref[...], axis=1)
    dn = (((1,), (1,)), ((), ()))
    pos_out[...] = lax.dot_general(pos_ref[...], cv, dn,
                                   preferred_element_type=jnp.float32)
    neg_out[...] = lax.dot_general(neg_ref[...], cv, dn,
                                   preferred_element_type=jnp.float32)


_tc_sims = pl.pallas_call(
    _tc_sim_body,
    grid=(B // BLK,),
    in_specs=[
        pl.BlockSpec((BLK, D), lambda i: (i, 0)),
        pl.BlockSpec((BLK, D), lambda i: (i, 0)),
        pl.BlockSpec((NUM_TAGS, ITEMS_PER_TAG, D), lambda i: (0, 0, 0)),
        pl.BlockSpec((NUM_TAGS, ITEMS_PER_TAG, 1), lambda i: (0, 0, 0)),
    ],
    out_specs=[
        pl.BlockSpec((BLK, D), lambda i: (i, 0)),
        pl.BlockSpec((BLK, D), lambda i: (i, 0)),
    ],
    out_shape=[
        jax.ShapeDtypeStruct((B, NUM_TAGS), jnp.float32),
        jax.ShapeDtypeStruct((B, NUM_TAGS), jnp.float32),
    ],
)


@jax.jit
def kernel(samples, neg_item, user_weight, item_weight, concept_rows,
           concept_cols, concept_vals):
    del concept_rows
    user_idx = samples[:, 0]
    pos_idx = samples[:, 1]
    user_embed, pos_rows, neg_rows, crows = _sc_gather(
        user_weight, item_weight, user_idx, pos_idx, neg_item, concept_cols)
    cw = crows.reshape(NUM_TAGS, ITEMS_PER_TAG, D)
    cvals = concept_vals.reshape(NUM_TAGS, ITEMS_PER_TAG, 1)
    pos_sim, neg_sim = _tc_sims(pos_rows, neg_rows, cw, cvals)
    return (user_embed, pos_sim, neg_sim)


# speedup vs baseline: 1.5158x; 1.5158x over previous
import jax
import jax.numpy as jnp
from jax import lax
from jax.experimental import pallas as pl
from jax.experimental.pallas import tpu as pltpu
from jax.experimental.pallas import tpu_sc as plsc

B = 16384
D = 64
ROWS_PER_TILE = 8
NUM_TAGS = 64
ITEMS_PER_TAG = 4
NC = 2
NS = 16
NW = NC * NS
B_PER_W = B // NW
C = NUM_TAGS * ITEMS_PER_TAG
C_PER_W = C // NW
K = 32
NCH = B_PER_W // K


def _sc_gather_body(user_w, item_w, user_idx, pos_idx, neg_idx,
                    cidx, user_out, pos_out, neg_out, crows_out,
                    iu_v, ip_v, in_v, ic_v, obuf0, obuf1, cobuf,
                    rsem0, rsem1, osem0, osem1, csem):
    wid = lax.axis_index("s") * NC + lax.axis_index("c")
    base = wid * B_PER_W
    cbase = wid * C_PER_W
    obufs = (obuf0, obuf1)
    rsems = (rsem0, rsem1)
    osems = (osem0, osem1)

    pltpu.sync_copy(user_idx.at[pl.ds(base, B_PER_W)], iu_v)
    pltpu.sync_copy(pos_idx.at[pl.ds(base, B_PER_W)], ip_v)
    pltpu.sync_copy(neg_idx.at[pl.ds(base, B_PER_W)], in_v)
    pltpu.sync_copy(cidx.at[pl.ds(cbase, C_PER_W)], ic_v.at[pl.ds(0, C_PER_W)])

    civ = ic_v[...]
    ctv = civ >> 3
    crv = civ & 7
    for jj in range(C_PER_W):
        pltpu.async_copy(item_w.at[ctv[jj], crv[jj]], cobuf.at[jj], csem)

    def fire(table, idx_v, c, par):
        for g in range(K // 16):
            iv = idx_v[pl.ds(c * K + g * 16, 16)]
            tiv = iv >> 3
            riv = iv & 7
            for jj in range(16):
                pltpu.async_copy(table.at[tiv[jj], riv[jj]],
                                 obufs[par].at[g * 16 + jj], rsems[par])

    def drain_rows(par):
        for g in range(K // ROWS_PER_TILE):
            pltpu.make_async_copy(
                user_w.at[0], obufs[par].at[pl.ds(g * ROWS_PER_TILE,
                                                  ROWS_PER_TILE)],
                rsems[par]).wait()

    def out_copy(out, c, par):
        return pltpu.async_copy(
            obufs[par], out.at[pl.ds(base + c * K, K)], osems[par])

    def out_drain(out, c, par):
        pltpu.make_async_copy(
            obufs[par], out.at[pl.ds(base + c * K, K)], osems[par]).wait()

    for table, idx_v, out in (
        (user_w, iu_v, user_out),
        (item_w, ip_v, pos_out),
        (item_w, in_v, neg_out),
    ):
        fire(table, idx_v, 0, 0)

        def body(c2, _, table=table, idx_v=idx_v, out=out):
            for par in range(2):
                c = c2 * 2 + par

                @pl.when(c + 1 < NCH)
                def _():
                    @pl.when(c >= 1)
                    def _():
                        out_drain(out, c - 1, 1 - par)

                    fire(table, idx_v, c + 1, 1 - par)

                drain_rows(par)
                out_copy(out, c, par)
            return 0

        lax.fori_loop(0, NCH // 2, body, 0)
        out_drain(out, NCH - 2, 0)
        out_drain(out, NCH - 1, 1)

    pltpu.make_async_copy(user_w.at[0], cobuf, csem).wait()
    pltpu.sync_copy(cobuf, crows_out.at[pl.ds(cbase, C_PER_W)])


_sc_gather = pl.kernel(
    _sc_gather_body,
    out_type=(
        jax.ShapeDtypeStruct((B, D), jnp.float32),
        jax.ShapeDtypeStruct((B, D), jnp.float32),
        jax.ShapeDtypeStruct((B, D), jnp.float32),
        jax.ShapeDtypeStruct((C, D), jnp.float32),
    ),
    mesh=plsc.VectorSubcoreMesh(core_axis_name="c", subcore_axis_name="s"),
    scratch_types=[
        pltpu.VMEM((B_PER_W,), jnp.int32),
        pltpu.VMEM((B_PER_W,), jnp.int32),
        pltpu.VMEM((B_PER_W,), jnp.int32),
        pltpu.VMEM((16,), jnp.int32),
        pltpu.VMEM((K, D), jnp.float32),
        pltpu.VMEM((K, D), jnp.float32),
        pltpu.VMEM((C_PER_W, D), jnp.float32),
        pltpu.SemaphoreType.DMA,
        pltpu.SemaphoreType.DMA,
        pltpu.SemaphoreType.DMA,
        pltpu.SemaphoreType.DMA,
        pltpu.SemaphoreType.DMA,
    ],
)


BLK = 2048


def _tc_sim_body(pos_ref, neg_ref, cw_ref, cv_ref, pos_out, neg_out):
    cv = jnp.sum(cw_ref[...] * cv_ref[...], axis=1)
    dn = (((1,), (1,)), ((), ()))
    pos_out[...] = lax.dot_general(cv, pos_ref[...], dn,
                                   preferred_element_type=jnp.float32)
    neg_out[...] = lax.dot_general(cv, neg_ref[...], dn,
                                   preferred_element_type=jnp.float32)


_tc_sims = pl.pallas_call(
    _tc_sim_body,
    grid=(B // BLK,),
    in_specs=[
        pl.BlockSpec((BLK, D), lambda i: (i, 0)),
        pl.BlockSpec((BLK, D), lambda i: (i, 0)),
        pl.BlockSpec((NUM_TAGS, ITEMS_PER_TAG, D), lambda i: (0, 0, 0)),
        pl.BlockSpec((NUM_TAGS, ITEMS_PER_TAG, 1), lambda i: (0, 0, 0)),
    ],
    out_specs=[
        pl.BlockSpec((NUM_TAGS, BLK), lambda i: (0, i)),
        pl.BlockSpec((NUM_TAGS, BLK), lambda i: (0, i)),
    ],
    out_shape=[
        jax.ShapeDtypeStruct((NUM_TAGS, B), jnp.float32),
        jax.ShapeDtypeStruct((NUM_TAGS, B), jnp.float32),
    ],
)


@jax.jit
def kernel(samples, neg_item, user_weight, item_weight, concept_rows,
           concept_cols, concept_vals):
    del concept_rows
    user_idx = samples[:, 0]
    pos_idx = samples[:, 1]
    user_w3 = user_weight.reshape(-1, ROWS_PER_TILE, D)
    item_w3 = item_weight.reshape(-1, ROWS_PER_TILE, D)
    user_embed, pos_rows, neg_rows, crows = _sc_gather(
        user_w3, item_w3, user_idx, pos_idx, neg_item, concept_cols)
    cw = crows.reshape(NUM_TAGS, ITEMS_PER_TAG, D)
    cvals = concept_vals.reshape(NUM_TAGS, ITEMS_PER_TAG, 1)
    pos_sim_t, neg_sim_t = _tc_sims(pos_rows, neg_rows, cw, cvals)
    return (user_embed, pos_sim_t.T, neg_sim_t.T)


# speedup vs baseline: 1.5310x; 1.0101x over previous
import jax
import jax.numpy as jnp
from jax import lax
from jax.experimental import pallas as pl
from jax.experimental.pallas import tpu as pltpu
from jax.experimental.pallas import tpu_sc as plsc

B = 16384
D = 64
ROWS_PER_TILE = 8
NUM_TAGS = 64
ITEMS_PER_TAG = 4
NC = 2
NS = 16
NW = NC * NS
B_PER_W = B // NW
C = NUM_TAGS * ITEMS_PER_TAG
C_PER_W = C // NW
K = 64
NCH = B_PER_W // K


def _sc_gather_body(user_w, item_w, user_idx, pos_idx, neg_idx,
                    cidx, user_out, pos_out, neg_out, crows_out,
                    iu_v, ip_v, in_v, ic_v, obuf0, obuf1, cobuf,
                    rsem0, rsem1, osem0, osem1, csem):
    wid = lax.axis_index("s") * NC + lax.axis_index("c")
    base = wid * B_PER_W
    cbase = wid * C_PER_W
    obufs = (obuf0, obuf1)
    rsems = (rsem0, rsem1)
    osems = (osem0, osem1)

    pltpu.sync_copy(user_idx.at[pl.ds(base, B_PER_W)], iu_v)
    pltpu.sync_copy(pos_idx.at[pl.ds(base, B_PER_W)], ip_v)
    pltpu.sync_copy(neg_idx.at[pl.ds(base, B_PER_W)], in_v)
    pltpu.sync_copy(cidx.at[pl.ds(cbase, C_PER_W)], ic_v.at[pl.ds(0, C_PER_W)])

    civ = ic_v[...]
    ctv = civ >> 3
    crv = civ & 7
    for jj in range(C_PER_W):
        pltpu.async_copy(item_w.at[ctv[jj], crv[jj]], cobuf.at[jj], csem)

    def fire(table, idx_v, c, par):
        for g in range(K // 16):
            iv = idx_v[pl.ds(c * K + g * 16, 16)]
            tiv = iv >> 3
            riv = iv & 7
            for jj in range(16):
                pltpu.async_copy(table.at[tiv[jj], riv[jj]],
                                 obufs[par].at[g * 16 + jj], rsems[par])

    def drain_rows(par):
        for g in range(K // ROWS_PER_TILE):
            pltpu.make_async_copy(
                user_w.at[0], obufs[par].at[pl.ds(g * ROWS_PER_TILE,
                                                  ROWS_PER_TILE)],
                rsems[par]).wait()

    def out_copy(out, c, par):
        return pltpu.async_copy(
            obufs[par], out.at[pl.ds(base + c * K, K)], osems[par])

    def out_drain(out, c, par):
        pltpu.make_async_copy(
            obufs[par], out.at[pl.ds(base + c * K, K)], osems[par]).wait()

    for table, idx_v, out in (
        (user_w, iu_v, user_out),
        (item_w, ip_v, pos_out),
        (item_w, in_v, neg_out),
    ):
        fire(table, idx_v, 0, 0)

        def body(c2, _, table=table, idx_v=idx_v, out=out):
            for par in range(2):
                c = c2 * 2 + par

                @pl.when(c + 1 < NCH)
                def _():
                    @pl.when(c >= 1)
                    def _():
                        out_drain(out, c - 1, 1 - par)

                    fire(table, idx_v, c + 1, 1 - par)

                drain_rows(par)
                out_copy(out, c, par)
            return 0

        lax.fori_loop(0, NCH // 2, body, 0)
        out_drain(out, NCH - 2, 0)
        out_drain(out, NCH - 1, 1)

    pltpu.make_async_copy(user_w.at[0], cobuf, csem).wait()
    pltpu.sync_copy(cobuf, crows_out.at[pl.ds(cbase, C_PER_W)])


_sc_gather = pl.kernel(
    _sc_gather_body,
    out_type=(
        jax.ShapeDtypeStruct((B, D), jnp.float32),
        jax.ShapeDtypeStruct((B, D), jnp.float32),
        jax.ShapeDtypeStruct((B, D), jnp.float32),
        jax.ShapeDtypeStruct((C, D), jnp.float32),
    ),
    mesh=plsc.VectorSubcoreMesh(core_axis_name="c", subcore_axis_name="s"),
    scratch_types=[
        pltpu.VMEM((B_PER_W,), jnp.int32),
        pltpu.VMEM((B_PER_W,), jnp.int32),
        pltpu.VMEM((B_PER_W,), jnp.int32),
        pltpu.VMEM((16,), jnp.int32),
        pltpu.VMEM((K, D), jnp.float32),
        pltpu.VMEM((K, D), jnp.float32),
        pltpu.VMEM((C_PER_W, D), jnp.float32),
        pltpu.SemaphoreType.DMA,
        pltpu.SemaphoreType.DMA,
        pltpu.SemaphoreType.DMA,
        pltpu.SemaphoreType.DMA,
        pltpu.SemaphoreType.DMA,
    ],
)


BLK = 2048


def _tc_sim_body(pos_ref, neg_ref, cw_ref, cv_ref, pos_out, neg_out):
    cv = jnp.sum(cw_ref[...] * cv_ref[...], axis=1)
    dn = (((1,), (1,)), ((), ()))
    pos_out[...] = lax.dot_general(cv, pos_ref[...], dn,
                                   preferred_element_type=jnp.float32)
    neg_out[...] = lax.dot_general(cv, neg_ref[...], dn,
                                   preferred_element_type=jnp.float32)


_tc_sims = pl.pallas_call(
    _tc_sim_body,
    grid=(B // BLK,),
    in_specs=[
        pl.BlockSpec((BLK, D), lambda i: (i, 0)),
        pl.BlockSpec((BLK, D), lambda i: (i, 0)),
        pl.BlockSpec((NUM_TAGS, ITEMS_PER_TAG, D), lambda i: (0, 0, 0)),
        pl.BlockSpec((NUM_TAGS, ITEMS_PER_TAG, 1), lambda i: (0, 0, 0)),
    ],
    out_specs=[
        pl.BlockSpec((NUM_TAGS, BLK), lambda i: (0, i)),
        pl.BlockSpec((NUM_TAGS, BLK), lambda i: (0, i)),
    ],
    out_shape=[
        jax.ShapeDtypeStruct((NUM_TAGS, B), jnp.float32),
        jax.ShapeDtypeStruct((NUM_TAGS, B), jnp.float32),
    ],
)


@jax.jit
def kernel(samples, neg_item, user_weight, item_weight, concept_rows,
           concept_cols, concept_vals):
    del concept_rows
    user_idx = samples[:, 0]
    pos_idx = samples[:, 1]
    user_w3 = user_weight.reshape(-1, ROWS_PER_TILE, D)
    item_w3 = item_weight.reshape(-1, ROWS_PER_TILE, D)
    user_embed, pos_rows, neg_rows, crows = _sc_gather(
        user_w3, item_w3, user_idx, pos_idx, neg_item, concept_cols)
    cw = crows.reshape(NUM_TAGS, ITEMS_PER_TAG, D)
    cvals = concept_vals.reshape(NUM_TAGS, ITEMS_PER_TAG, 1)
    pos_sim_t, neg_sim_t = _tc_sims(pos_rows, neg_rows, cw, cvals)
    return (user_embed, pos_sim_t.T, neg_sim_t.T)


# speedup vs baseline: 1.5457x; 1.0096x over previous
import jax
import jax.numpy as jnp
from jax import lax
from jax.experimental import pallas as pl
from jax.experimental.pallas import tpu as pltpu
from jax.experimental.pallas import tpu_sc as plsc

B = 16384
D = 64
ROWS_PER_TILE = 8
NUM_TAGS = 64
ITEMS_PER_TAG = 4
NC = 2
NS = 16
NW = NC * NS
B_PER_W = B // NW
C = NUM_TAGS * ITEMS_PER_TAG
C_PER_W = C // NW
K = 64
NCH = B_PER_W // K


def _run_stream(table, idx_v, out, base, obufs, rsems, osems):
    def fire(c, par):
        for g in range(K // 16):
            iv = idx_v[pl.ds(c * K + g * 16, 16)]
            tiv = iv >> 3
            riv = iv & 7
            for jj in range(16):
                pltpu.async_copy(table.at[tiv[jj], riv[jj]],
                                 obufs[par].at[g * 16 + jj], rsems[par])

    def drain_rows(par):
        for g in range(K // ROWS_PER_TILE):
            pltpu.make_async_copy(
                table.at[0], obufs[par].at[pl.ds(g * ROWS_PER_TILE,
                                                 ROWS_PER_TILE)],
                rsems[par]).wait()

    def out_copy(c, par):
        pltpu.async_copy(
            obufs[par], out.at[pl.ds(base + c * K, K)], osems[par])

    def out_drain(c, par):
        pltpu.make_async_copy(
            obufs[par], out.at[pl.ds(base + c * K, K)], osems[par]).wait()

    fire(0, 0)

    def body(c2, _):
        for par in range(2):
            c = c2 * 2 + par

            @pl.when(c + 1 < NCH)
            def _():
                @pl.when(c >= 1)
                def _():
                    out_drain(c - 1, 1 - par)

                fire(c + 1, 1 - par)

            drain_rows(par)
            out_copy(c, par)
        return 0

    lax.fori_loop(0, NCH // 2, body, 0)
    out_drain(NCH - 2, 0)
    out_drain(NCH - 1, 1)


def _sc_item_body(item_w, pos_idx, neg_idx, cidx,
                  pos_out, neg_out, crows_out,
                  ip_v, in_v, ic_v, obuf0, obuf1, cobuf,
                  rsem0, rsem1, osem0, osem1, csem):
    wid = lax.axis_index("s") * NC + lax.axis_index("c")
    base = wid * B_PER_W
    cbase = wid * C_PER_W

    pltpu.sync_copy(pos_idx.at[pl.ds(base, B_PER_W)], ip_v)
    pltpu.sync_copy(neg_idx.at[pl.ds(base, B_PER_W)], in_v)
    pltpu.sync_copy(cidx.at[pl.ds(cbase, C_PER_W)], ic_v.at[pl.ds(0, C_PER_W)])

    civ = ic_v[...]
    ctv = civ >> 3
    crv = civ & 7
    for jj in range(C_PER_W):
        pltpu.async_copy(item_w.at[ctv[jj], crv[jj]], cobuf.at[jj], csem)

    _run_stream(item_w, ip_v, pos_out, base,
                (obuf0, obuf1), (rsem0, rsem1), (osem0, osem1))
    _run_stream(item_w, in_v, neg_out, base,
                (obuf0, obuf1), (rsem0, rsem1), (osem0, osem1))

    pltpu.make_async_copy(item_w.at[0], cobuf, csem).wait()
    pltpu.sync_copy(cobuf, crows_out.at[pl.ds(cbase, C_PER_W)])


def _sc_user_body(user_w, user_idx, user_out,
                  iu_v, obuf0, obuf1, rsem0, rsem1, osem0, osem1):
    wid = lax.axis_index("s") * NC + lax.axis_index("c")
    base = wid * B_PER_W
    pltpu.sync_copy(user_idx.at[pl.ds(base, B_PER_W)], iu_v)
    _run_stream(user_w, iu_v, user_out, base,
                (obuf0, obuf1), (rsem0, rsem1), (osem0, osem1))


_MESH = plsc.VectorSubcoreMesh(core_axis_name="c", subcore_axis_name="s")

_sc_item = pl.kernel(
    _sc_item_body,
    out_type=(
        jax.ShapeDtypeStruct((B, D), jnp.float32),
        jax.ShapeDtypeStruct((B, D), jnp.float32),
        jax.ShapeDtypeStruct((C, D), jnp.float32),
    ),
    mesh=_MESH,
    scratch_types=[
        pltpu.VMEM((B_PER_W,), jnp.int32),
        pltpu.VMEM((B_PER_W,), jnp.int32),
        pltpu.VMEM((16,), jnp.int32),
        pltpu.VMEM((K, D), jnp.float32),
        pltpu.VMEM((K, D), jnp.float32),
        pltpu.VMEM((C_PER_W, D), jnp.float32),
        pltpu.SemaphoreType.DMA,
        pltpu.SemaphoreType.DMA,
        pltpu.SemaphoreType.DMA,
        pltpu.SemaphoreType.DMA,
        pltpu.SemaphoreType.DMA,
    ],
)

_sc_user = pl.kernel(
    _sc_user_body,
    out_type=jax.ShapeDtypeStruct((B, D), jnp.float32),
    mesh=_MESH,
    scratch_types=[
        pltpu.VMEM((B_PER_W,), jnp.int32),
        pltpu.VMEM((K, D), jnp.float32),
        pltpu.VMEM((K, D), jnp.float32),
        pltpu.SemaphoreType.DMA,
        pltpu.SemaphoreType.DMA,
        pltpu.SemaphoreType.DMA,
        pltpu.SemaphoreType.DMA,
    ],
)


BLK = 2048


def _tc_sim_body(pos_ref, neg_ref, cw_ref, cv_ref, pos_out, neg_out):
    cv = jnp.sum(cw_ref[...] * cv_ref[...], axis=1)
    dn = (((1,), (1,)), ((), ()))
    pos_out[...] = lax.dot_general(cv, pos_ref[...], dn,
                                   preferred_element_type=jnp.float32)
    neg_out[...] = lax.dot_general(cv, neg_ref[...], dn,
                                   preferred_element_type=jnp.float32)


_tc_sims = pl.pallas_call(
    _tc_sim_body,
    grid=(B // BLK,),
    in_specs=[
        pl.BlockSpec((BLK, D), lambda i: (i, 0)),
        pl.BlockSpec((BLK, D), lambda i: (i, 0)),
        pl.BlockSpec((NUM_TAGS, ITEMS_PER_TAG, D), lambda i: (0, 0, 0)),
        pl.BlockSpec((NUM_TAGS, ITEMS_PER_TAG, 1), lambda i: (0, 0, 0)),
    ],
    out_specs=[
        pl.BlockSpec((NUM_TAGS, BLK), lambda i: (0, i)),
        pl.BlockSpec((NUM_TAGS, BLK), lambda i: (0, i)),
    ],
    out_shape=[
        jax.ShapeDtypeStruct((NUM_TAGS, B), jnp.float32),
        jax.ShapeDtypeStruct((NUM_TAGS, B), jnp.float32),
    ],
)


@jax.jit
def kernel(samples, neg_item, user_weight, item_weight, concept_rows,
           concept_cols, concept_vals):
    del concept_rows
    user_idx = samples[:, 0]
    pos_idx = samples[:, 1]
    user_w3 = user_weight.reshape(-1, ROWS_PER_TILE, D)
    item_w3 = item_weight.reshape(-1, ROWS_PER_TILE, D)
    pos_rows, neg_rows, crows = _sc_item(item_w3, pos_idx, neg_item,
                                         concept_cols)
    user_embed = _sc_user(user_w3, user_idx)
    cw = crows.reshape(NUM_TAGS, ITEMS_PER_TAG, D)
    cvals = concept_vals.reshape(NUM_TAGS, ITEMS_PER_TAG, 1)
    pos_sim_t, neg_sim_t = _tc_sims(pos_rows, neg_rows, cw, cvals)
    return (user_embed, pos_sim_t.T, neg_sim_t.T)
